# baseline (speedup 1.0000x reference)
import jax
import jax.numpy as jnp
from jax import lax
from jax.experimental import pallas as pl
from jax.experimental.pallas import tpu as pltpu
from jax.experimental.pallas import tpu_sc as plsc

BATCH = 4096
SEQ = 200
EMBED = 64
NC, NS, LANES = 2, 16, 16
NW = NC * NS
BROWS = BATCH // NW
IDX_PER_W = BROWS * SEQ


def _body(idx_hbm, tok_hbm, pos_hbm, out_hbm,
          idx_v, idx_t, pos_v, gbufs, stage_a, stage_b,
          gsems, osems):
    wid = lax.axis_index("s") * NC + lax.axis_index("c")
    base = wid * IDX_PER_W
    pltpu.sync_copy(idx_hbm.at[pl.ds(base, IDX_PER_W)], idx_v)
    pltpu.sync_copy(pos_hbm, pos_v)

    iota = lax.iota(jnp.int32, LANES)
    iota_seq = iota * SEQ

    @pl.loop(0, SEQ)
    def _tr(s):
        for g in range(BROWS // LANES):
            vals = plsc.load_gather(idx_v, [iota_seq + (g * LANES * SEQ + s)])
            idx_t[s, pl.ds(g * LANES, LANES)] = vals

    stages = (stage_a, stage_b)
    e_hi = [(iota + c * LANES) // 8 for c in range(EMBED // LANES)]
    e_lo = [(iota + c * LANES) % 8 for c in range(EMBED // LANES)]

    def gather(s, b):
        pltpu.async_copy(tok_hbm.at[idx_t.at[s]], gbufs.at[b], gsems[b])

    def gather_wait(s, b):
        pltpu.make_async_copy(
            tok_hbm.at[idx_t.at[s]], gbufs.at[b], gsems[b]
        ).wait()

    def put(s, b):
        pltpu.async_copy(stages[b], out_hbm.at[s, :, wid], osems[b])

    def put_wait(s, b):
        pltpu.make_async_copy(
            stages[b], out_hbm.at[s, :, wid], osems[b]
        ).wait()

    gather(0, 0)

    @pl.loop(0, SEQ, step=2)
    def _outer(k0):
        for bi in range(2):
            s = k0 + bi
            stage = stages[bi]
            gather_wait(s, bi)

            @pl.when(s + 1 < SEQ)
            def _prefetch():
                gather(s + 1, 1 - bi)

            @pl.when(s >= 2)
            def _drain():
                put_wait(s - 2, bi)

            @pl.loop(0, BROWS)
            def _row(r):
                rsplat = jnp.full((LANES,), 0, jnp.int32) + r
                for c in range(EMBED // LANES):
                    v = gbufs[bi, r, pl.ds(c * LANES, LANES)] \
                        + pos_v[s, pl.ds(c * LANES, LANES)]
                    plsc.store_scatter(stage, [e_hi[c], e_lo[c], rsplat], v)

            put(s, bi)

    put_wait(SEQ - 2, 0)
    put_wait(SEQ - 1, 1)


def kernel(inputs, token_table, pos_table):
    flat_idx = inputs.reshape(-1).astype(jnp.int32)
    mesh = plsc.VectorSubcoreMesh(core_axis_name="c", subcore_axis_name="s")
    out = pl.kernel(
        _body,
        out_type=jax.ShapeDtypeStruct((SEQ, EMBED // 8, NW, 8, 128), jnp.float32),
        mesh=mesh,
        scratch_types=[
            pltpu.VMEM((IDX_PER_W,), jnp.int32),
            pltpu.VMEM((SEQ, BROWS), jnp.int32),
            pltpu.VMEM((SEQ, EMBED), jnp.float32),
            pltpu.VMEM((2, BROWS, EMBED), jnp.float32),
            pltpu.VMEM((EMBED // 8, 8, 128), jnp.float32),
            pltpu.VMEM((EMBED // 8, 8, 128), jnp.float32),
            [pltpu.SemaphoreType.DMA] * 2,
            [pltpu.SemaphoreType.DMA] * 2,
        ],
        compiler_params=pltpu.CompilerParams(
            use_tc_tiling_on_sc=False, needs_layout_passes=False
        ),
    )(flat_idx, token_table, pos_table)
    return out.transpose(2, 4, 0, 1, 3).reshape(BATCH, SEQ, EMBED)


# speedup vs baseline: 2.0189x; 2.0189x over previous
import jax
import jax.numpy as jnp
from jax import lax
from jax.experimental import pallas as pl
from jax.experimental.pallas import tpu as pltpu
from jax.experimental.pallas import tpu_sc as plsc

BATCH = 4096
SEQ = 200
EMBED = 64
NC, NS, LANES = 2, 16, 16
NW = NC * NS
BROWS = BATCH // NW
IDX_PER_W = BROWS * SEQ


def _body(idx_hbm, tok_hbm, pos_hbm, out_hbm,
          idx_v, idx_t, pos_v, gbufs, stage_a, stage_b,
          gsems, osems):
    wid = lax.axis_index("s") * NC + lax.axis_index("c")
    base = wid * IDX_PER_W
    pltpu.sync_copy(idx_hbm.at[pl.ds(base, IDX_PER_W)], idx_v)
    pltpu.sync_copy(pos_hbm, pos_v)

    iota = lax.iota(jnp.int32, LANES)
    iota_seq = iota * SEQ

    @pl.loop(0, SEQ)
    def _tr(s):
        for g in range(BROWS // LANES):
            vals = plsc.load_gather(idx_v, [iota_seq + (g * LANES * SEQ + s)])
            idx_t[s, pl.ds(g * LANES, LANES)] = vals

    stages = (stage_a, stage_b)

    def gather(s, b):
        pltpu.async_copy(tok_hbm.at[idx_t.at[s]], gbufs.at[b], gsems[b])

    def gather_wait(s, b):
        pltpu.make_async_copy(
            tok_hbm.at[idx_t.at[s]], gbufs.at[b], gsems[b]
        ).wait()

    def put(s, b):
        pltpu.async_copy(stages[b], out_hbm.at[s, :, wid], osems[b])

    def put_wait(s, b):
        pltpu.make_async_copy(
            stages[b], out_hbm.at[s, :, wid], osems[b]
        ).wait()

    gather(0, 0)

    @pl.loop(0, SEQ, step=2)
    def _outer(k0):
        for bi in range(2):
            s = k0 + bi
            stage = stages[bi]
            gather_wait(s, bi)

            @pl.when(s + 1 < SEQ)
            def _prefetch():
                gather(s + 1, 1 - bi)

            @pl.when(s >= 2)
            def _drain():
                put_wait(s - 2, bi)

            s_splat = jnp.full((LANES,), 0, jnp.int32) + s
            bi_splat = jnp.full((LANES,), bi, jnp.int32)

            @pl.loop(0, LANES)
            def _diag(d):
                rot = (iota + d) & (LANES - 1)
                for e0 in range(0, EMBED, LANES):
                    e_idx = e0 + rot
                    pv = plsc.load_gather(pos_v, [s_splat, e_idx])
                    i_hi = e_idx // 8
                    i_lo = e_idx % 8
                    for r0 in range(0, BROWS, LANES):
                        b_idx = iota + r0
                        tv = plsc.load_gather(gbufs, [bi_splat, b_idx, e_idx])
                        plsc.store_scatter(stage, [i_hi, i_lo, b_idx], tv + pv)

            put(s, bi)

    put_wait(SEQ - 2, 0)
    put_wait(SEQ - 1, 1)


def kernel(inputs, token_table, pos_table):
    flat_idx = inputs.reshape(-1).astype(jnp.int32)
    mesh = plsc.VectorSubcoreMesh(core_axis_name="c", subcore_axis_name="s")
    out = pl.kernel(
        _body,
        out_type=jax.ShapeDtypeStruct((SEQ, EMBED // 8, NW, 8, 128), jnp.float32),
        mesh=mesh,
        scratch_types=[
            pltpu.VMEM((IDX_PER_W,), jnp.int32),
            pltpu.VMEM((SEQ, BROWS), jnp.int32),
            pltpu.VMEM((SEQ, EMBED), jnp.float32),
            pltpu.VMEM((2, BROWS, EMBED), jnp.float32),
            pltpu.VMEM((EMBED // 8, 8, 128), jnp.float32),
            pltpu.VMEM((EMBED // 8, 8, 128), jnp.float32),
            [pltpu.SemaphoreType.DMA] * 2,
            [pltpu.SemaphoreType.DMA] * 2,
        ],
        compiler_params=pltpu.CompilerParams(
            use_tc_tiling_on_sc=False, needs_layout_passes=False
        ),
    )(flat_idx, token_table, pos_table)
    return out.transpose(2, 4, 0, 1, 3).reshape(BATCH, SEQ, EMBED)
